# baseline (speedup 1.0000x reference)
import functools

import jax
import jax.numpy as jnp
from jax import lax
from jax.experimental import pallas as pl
from jax.experimental.pallas import tpu as pltpu
from jax.experimental.pallas import tpu_sc as plsc

_LANES = 16
_NUM_WORKERS = 32
_T = 256
_IDXW = 128


@functools.partial(jax.jit, static_argnums=(10, 11, 12))
def _sc_embed(x, ted, tsumo, ri, called, table, tb, rb, tsb, clb, N, D, V):
    KD = D // _LANES
    JN = _T // _IDXW
    tok_per_w = N // _NUM_WORKERS
    chunks = tok_per_w // _T

    mesh = plsc.VectorSubcoreMesh(core_axis_name="c", subcore_axis_name="s")

    @functools.partial(
        pl.kernel,
        out_type=jax.ShapeDtypeStruct((N, D), jnp.float32),
        mesh=mesh,
        scratch_types=[
            pltpu.VMEM((V, D), jnp.float32),
            pltpu.VMEM_SHARED((V, D), jnp.float32),
            pltpu.VMEM((4, D), jnp.float32),
            pltpu.VMEM((3, JN, _IDXW), jnp.int32),
            pltpu.VMEM((3, 4, _T), jnp.float32),
            pltpu.VMEM((3, _T, D), jnp.float32),
            pltpu.SemaphoreType.DMA((3,)),
            pltpu.SemaphoreType.DMA,
            pltpu.SemaphoreType.DMA((3,)),
        ],
    )
    def k(x_hbm, ted_hbm, tsumo_hbm, ri_hbm, called_hbm, table_hbm,
          tb_hbm, rb_hbm, tsb_hbm, clb_hbm, out_hbm,
          table_v, table_sp, bias_v, x_v, sc_v, out_v,
          sem_in, sem_g, sem_out):
        wid = lax.axis_index("s") * 2 + lax.axis_index("c")

        pltpu.sync_copy(table_hbm, table_v)
        pltpu.sync_copy(tb_hbm, bias_v.at[0])
        pltpu.sync_copy(rb_hbm, bias_v.at[1])
        pltpu.sync_copy(tsb_hbm, bias_v.at[2])
        pltpu.sync_copy(clb_hbm, bias_v.at[3])

        cs = [bias_v[2, pl.ds(kk * _LANES, _LANES)]
              + bias_v[3, pl.ds(kk * _LANES, _LANES)] for kk in range(KD)]

        def fold(j, _):
            for kk in range(KD):
                sl = pl.ds(kk * _LANES, _LANES)
                table_v[j, sl] = table_v[j, sl] + cs[kk]
            return 0
        lax.fori_loop(0, V, fold, 0)

        @pl.when(lax.axis_index("s") == 0)
        def _pub():
            pltpu.sync_copy(table_v, table_sp)
        plsc.subcore_barrier()

        tbs = [bias_v[0, pl.ds(kk * _LANES, _LANES)] for kk in range(KD)]
        rbs = [bias_v[1, pl.ds(kk * _LANES, _LANES)] for kk in range(KD)]

        def in_copies(c, b):
            tok0 = pl.multiple_of(wid * tok_per_w + c * _T, _T)
            row0 = pl.multiple_of((wid * tok_per_w + c * _T) // _IDXW, JN)
            return [
                pltpu.make_async_copy(
                    x_hbm.at[pl.ds(row0, JN), :], x_v.at[b], sem_in.at[b]),
                pltpu.make_async_copy(
                    ted_hbm.at[pl.ds(tok0, _T)], sc_v.at[b, 0], sem_in.at[b]),
                pltpu.make_async_copy(
                    tsumo_hbm.at[pl.ds(tok0, _T)], sc_v.at[b, 1], sem_in.at[b]),
                pltpu.make_async_copy(
                    ri_hbm.at[pl.ds(tok0, _T)], sc_v.at[b, 2], sem_in.at[b]),
                pltpu.make_async_copy(
                    called_hbm.at[pl.ds(tok0, _T)], sc_v.at[b, 3], sem_in.at[b]),
            ]

        def gather_copies(b):
            return [
                pltpu.make_async_copy(
                    table_sp.at[x_v.at[b, j]],
                    out_v.at[b, pl.ds(j * _IDXW, _IDXW)], sem_g)
                for j in range(JN)
            ]

        def out_copy(c, b):
            tok0 = pl.multiple_of(wid * tok_per_w + c * _T, _T)
            return pltpu.make_async_copy(
                out_v.at[b], out_hbm.at[pl.ds(tok0, _T)], sem_out.at[b])

        for cp in in_copies(0, 0):
            cp.start()
        for cp in in_copies(0, 0):
            cp.wait()
        for cp in gather_copies(0):
            cp.start()
        for cp in in_copies(1, 1):
            cp.start()

        def triple(cc, _):
            for b in range(3):
                c = cc * 3 + b

                @pl.when(c < chunks)
                def _chunk():
                    for cp in gather_copies(b):
                        cp.wait()

                    bn = (b + 1) % 3

                    @pl.when(c + 1 < chunks)
                    def _next_gather():
                        for cp in in_copies(c + 1, bn):
                            cp.wait()

                        @pl.when(c >= 2)
                        def _free_buf():
                            out_copy(c - 2, bn).wait()

                        for cp in gather_copies(bn):
                            cp.start()

                    @plsc.parallel_loop(0, _T // _LANES, 1, unroll=4)
                    def _grp(g):
                        t0 = pl.multiple_of(g * _LANES, _LANES)
                        ted16 = sc_v[b, 0, pl.ds(t0, _LANES)]
                        ri16 = sc_v[b, 2, pl.ds(t0, _LANES)]
                        st16 = (sc_v[b, 1, pl.ds(t0, _LANES)]
                                + sc_v[b, 3, pl.ds(t0, _LANES)])
                        for lane in range(_LANES):
                            tedt = ted16[lane]
                            rit = ri16[lane]
                            st = st16[lane]
                            vals = [tedt * tbs[kk] + (rit * rbs[kk] + st)
                                    for kk in range(KD)]
                            for kk in range(KD):
                                plsc.addupdate(
                                    out_v.at[b, t0 + lane,
                                             pl.ds(kk * _LANES, _LANES)],
                                    vals[kk])

                    out_copy(c, b).start()

                    @pl.when(c + 2 < chunks)
                    def _next_inputs():
                        for cp in in_copies(c + 2, (b + 2) % 3):
                            cp.start()
            return 0
        lax.fori_loop(0, (chunks + 2) // 3, triple, 0)

        out_copy(chunks - 2, (chunks - 2) % 3).wait()
        out_copy(chunks - 1, (chunks - 1) % 3).wait()

    return k(x, ted, tsumo, ri, called, table, tb, rb, tsb, clb)


def kernel(x, tedashi, tsumogiri, riichi, called, table,
           tedashi_bias, tsumogiri_bias, riichi_bias, called_bias):
    B, L = x.shape
    V, D = table.shape
    N = B * L
    out = _sc_embed(
        x.reshape(N // _IDXW, _IDXW).astype(jnp.int32),
        tedashi.reshape(N), tsumogiri.reshape(N),
        riichi.reshape(N), called.reshape(N),
        table,
        tedashi_bias.reshape(D), riichi_bias.reshape(D),
        tsumogiri_bias.reshape(D), called_bias.reshape(D),
        N, D, V)
    return out.reshape(B, L, D)


# speedup vs baseline: 2.2966x; 2.2966x over previous
import functools

import jax
import jax.numpy as jnp
from jax import lax
from jax.experimental import pallas as pl
from jax.experimental.pallas import tpu as pltpu
from jax.experimental.pallas import tpu_sc as plsc

_LANES = 16
_NUM_WORKERS = 32
_T = 256
_TB = 512
_VPAD = 64


def _sc_embed(x, ted, tsumo, ri, called, table, tb, rb, tsb, clb, N, NS, D, V):
    KD = D // _LANES
    tok_per_w = NS // _NUM_WORKERS
    chunks = tok_per_w // _T
    assert chunks % 2 == 0

    mesh = plsc.VectorSubcoreMesh(core_axis_name="c", subcore_axis_name="s")

    @functools.partial(
        pl.kernel,
        out_type=jax.ShapeDtypeStruct((N, D), jnp.float32),
        mesh=mesh,
        scratch_types=[
            pltpu.VMEM((V * D,), jnp.float32),
            pltpu.VMEM((4, D), jnp.float32),
            pltpu.VMEM((2, _T), jnp.int32),
            pltpu.VMEM((2, 4, _T), jnp.float32),
            pltpu.VMEM((2, _T, D), jnp.float32),
            pltpu.SemaphoreType.DMA((2,)),
            pltpu.SemaphoreType.DMA((2,)),
        ],
    )
    def k(x_hbm, ted_hbm, tsumo_hbm, ri_hbm, called_hbm, table_hbm,
          tb_hbm, rb_hbm, tsb_hbm, clb_hbm, out_hbm,
          table_v, bias_v, x_v, sc_v, out_v, sem_in, sem_out):
        wid = lax.axis_index("s") * 2 + lax.axis_index("c")

        pltpu.sync_copy(table_hbm, table_v)
        pltpu.sync_copy(tb_hbm, bias_v.at[0])
        pltpu.sync_copy(rb_hbm, bias_v.at[1])
        pltpu.sync_copy(tsb_hbm, bias_v.at[2])
        pltpu.sync_copy(clb_hbm, bias_v.at[3])

        def in_copies(c, b):
            tok0 = pl.multiple_of(wid * tok_per_w + c * _T, _T)
            return [
                pltpu.make_async_copy(
                    x_hbm.at[pl.ds(tok0, _T)], x_v.at[b], sem_in.at[b]),
                pltpu.make_async_copy(
                    ted_hbm.at[pl.ds(tok0, _T)], sc_v.at[b, 0], sem_in.at[b]),
                pltpu.make_async_copy(
                    tsumo_hbm.at[pl.ds(tok0, _T)], sc_v.at[b, 1], sem_in.at[b]),
                pltpu.make_async_copy(
                    ri_hbm.at[pl.ds(tok0, _T)], sc_v.at[b, 2], sem_in.at[b]),
                pltpu.make_async_copy(
                    called_hbm.at[pl.ds(tok0, _T)], sc_v.at[b, 3], sem_in.at[b]),
            ]

        def out_copy(c, b):
            tok0 = pl.multiple_of(wid * tok_per_w + c * _T, _T)
            return pltpu.make_async_copy(
                out_v.at[b], out_hbm.at[pl.ds(tok0, _T)], sem_out.at[b])

        cs = [bias_v[2, pl.ds(kk * _LANES, _LANES)]
              + bias_v[3, pl.ds(kk * _LANES, _LANES)] for kk in range(KD)]

        def fold(j, _):
            base = pl.multiple_of(j * D, D)
            for kk in range(KD):
                off = base + kk * _LANES
                table_v[pl.ds(off, _LANES)] = table_v[pl.ds(off, _LANES)] + cs[kk]
            return 0
        lax.fori_loop(0, V, fold, 0)

        tbs = [bias_v[0, pl.ds(kk * _LANES, _LANES)] for kk in range(KD)]
        rbs = [bias_v[1, pl.ds(kk * _LANES, _LANES)] for kk in range(KD)]

        for cp in in_copies(0, 0):
            cp.start()

        def pair(cc, _):
            for b in range(2):
                c = cc * 2 + b

                @pl.when(c + 1 < chunks)
                def _prefetch():
                    for cp in in_copies(c + 1, 1 - b):
                        cp.start()

                for cp in in_copies(c, b):
                    cp.wait()

                @pl.when(c >= 2)
                def _drain():
                    out_copy(c - 2, b).wait()

                @plsc.parallel_loop(0, _T // _LANES, 1, unroll=4)
                def _grp(g):
                    t0 = pl.multiple_of(g * _LANES, _LANES)
                    x16 = x_v[b, pl.ds(t0, _LANES)]
                    ted16 = sc_v[b, 0, pl.ds(t0, _LANES)]
                    ri16 = sc_v[b, 2, pl.ds(t0, _LANES)]
                    st16 = (sc_v[b, 1, pl.ds(t0, _LANES)]
                            + sc_v[b, 3, pl.ds(t0, _LANES)])
                    for lane in range(_LANES):
                        base = pl.multiple_of(x16[lane] * D, D)
                        tedt = ted16[lane]
                        rit = ri16[lane]
                        st = st16[lane]
                        rows = [table_v[pl.ds(base + kk * _LANES, _LANES)]
                                for kk in range(KD)]
                        vals = [(rows[kk] + st)
                                + (tedt * tbs[kk] + rit * rbs[kk])
                                for kk in range(KD)]
                        for kk in range(KD):
                            out_v[b, t0 + lane,
                                  pl.ds(kk * _LANES, _LANES)] = vals[kk]

                out_copy(c, b).start()
            return 0
        lax.fori_loop(0, chunks // 2, pair, 0)

        out_copy(chunks - 2, 0).wait()
        out_copy(chunks - 1, 1).wait()

    return k(x, ted, tsumo, ri, called, table, tb, rb, tsb, clb)


def _tc_embed(base, x, ted, tsumo, ri, called, table_pad,
              tb, rb, tsb, clb, N, NS, D):
    nb = (N - NS) // _TB
    blk0 = NS // _TB

    def body(base_ref, x_ref, ted_ref, tsumo_ref, ri_ref, called_ref,
             table_ref, tb_ref, rb_ref, tsb_ref, clb_ref, out_ref):
        ids = x_ref[0, 0, :]
        oh = (ids[:, None]
              == lax.broadcasted_iota(jnp.int32, (_TB, _VPAD), 1)
              ).astype(jnp.float32)
        emb = jnp.dot(oh, table_ref[...], preferred_element_type=jnp.float32)
        ted = ted_ref[0, 0, :][:, None]
        tsumo = tsumo_ref[0, 0, :][:, None]
        ri = ri_ref[0, 0, :][:, None]
        called = called_ref[0, 0, :][:, None]
        out_ref[...] = (emb + (tsumo + called)
                        + ted * tb_ref[...] + ri * rb_ref[...]
                        + (tsb_ref[...] + clb_ref[...]))

    tok_spec = pl.BlockSpec((1, 1, _TB), lambda i: (i, 0, 0))
    vec_spec = pl.BlockSpec((1, D), lambda i: (0, 0))
    return pl.pallas_call(
        body,
        grid=(nb,),
        in_specs=[
            pl.BlockSpec(memory_space=pl.ANY),
            tok_spec, tok_spec, tok_spec, tok_spec, tok_spec,
            pl.BlockSpec((_VPAD, D), lambda i: (0, 0)),
            vec_spec, vec_spec, vec_spec, vec_spec,
        ],
        out_specs=pl.BlockSpec((_TB, D), lambda i: (blk0 + i, 0)),
        out_shape=jax.ShapeDtypeStruct((N, D), jnp.float32),
        input_output_aliases={0: 0},
        compiler_params=pltpu.CompilerParams(
            dimension_semantics=("arbitrary",)),
    )(base, x, ted, tsumo, ri, called, table_pad, tb, rb, tsb, clb)


@functools.partial(jax.jit, static_argnums=(10, 11, 12, 13))
def _embed(x, ted, tsumo, ri, called, table, tb, rb, tsb, clb, N, NS, D, V):
    base = _sc_embed(x, ted, tsumo, ri, called, table.reshape(V * D),
                     tb.reshape(D), rb.reshape(D), tsb.reshape(D),
                     clb.reshape(D), N, NS, D, V)
    nb = (N - NS) // _TB
    table_pad = jnp.zeros((_VPAD, D), jnp.float32).at[:V].set(table)
    return _tc_embed(
        base,
        x[NS:].reshape(nb, 1, _TB),
        ted[NS:].reshape(nb, 1, _TB), tsumo[NS:].reshape(nb, 1, _TB),
        ri[NS:].reshape(nb, 1, _TB), called[NS:].reshape(nb, 1, _TB),
        table_pad, tb.reshape(1, D), rb.reshape(1, D),
        tsb.reshape(1, D), clb.reshape(1, D), N, NS, D)


def kernel(x, tedashi, tsumogiri, riichi, called, table,
           tedashi_bias, tsumogiri_bias, riichi_bias, called_bias):
    B, L = x.shape
    V, D = table.shape
    N = B * L
    NS = N // 2
    out = _embed(
        x.reshape(N).astype(jnp.int32),
        tedashi.reshape(N), tsumogiri.reshape(N),
        riichi.reshape(N), called.reshape(N),
        table,
        tedashi_bias.reshape(D), riichi_bias.reshape(D),
        tsumogiri_bias.reshape(D), called_bias.reshape(D),
        N, NS, D, V)
    return out.reshape(B, L, D)


# speedup vs baseline: 3.3842x; 1.4736x over previous
import functools

import jax
import jax.numpy as jnp
from jax import lax
from jax.experimental import pallas as pl
from jax.experimental.pallas import tpu as pltpu
from jax.experimental.pallas import tpu_sc as plsc

_LANES = 16
_NUM_WORKERS = 32
_T = 256
_TB = 8192
_VPAD = 64


def _sc_embed(x, ted, tsumo, ri, called, table, tb, rb, tsb, clb, N, NS, D, V):
    KD = D // _LANES
    tok_per_w = NS // _NUM_WORKERS
    chunks = tok_per_w // _T
    assert chunks % 2 == 0

    mesh = plsc.VectorSubcoreMesh(core_axis_name="c", subcore_axis_name="s")

    @functools.partial(
        pl.kernel,
        out_type=jax.ShapeDtypeStruct((N, D), jnp.float32),
        mesh=mesh,
        scratch_types=[
            pltpu.VMEM((V * D,), jnp.float32),
            pltpu.VMEM((4, D), jnp.float32),
            pltpu.VMEM((2, _T), jnp.int32),
            pltpu.VMEM((2, 4, _T), jnp.float32),
            pltpu.VMEM((2, _T, D), jnp.float32),
            pltpu.SemaphoreType.DMA((2,)),
            pltpu.SemaphoreType.DMA((2,)),
        ],
    )
    def k(x_hbm, ted_hbm, tsumo_hbm, ri_hbm, called_hbm, table_hbm,
          tb_hbm, rb_hbm, tsb_hbm, clb_hbm, out_hbm,
          table_v, bias_v, x_v, sc_v, out_v, sem_in, sem_out):
        wid = lax.axis_index("s") * 2 + lax.axis_index("c")

        pltpu.sync_copy(table_hbm, table_v)
        pltpu.sync_copy(tb_hbm, bias_v.at[0])
        pltpu.sync_copy(rb_hbm, bias_v.at[1])
        pltpu.sync_copy(tsb_hbm, bias_v.at[2])
        pltpu.sync_copy(clb_hbm, bias_v.at[3])

        def in_copies(c, b):
            tok0 = pl.multiple_of(wid * tok_per_w + c * _T, _T)
            return [
                pltpu.make_async_copy(
                    x_hbm.at[pl.ds(tok0, _T)], x_v.at[b], sem_in.at[b]),
                pltpu.make_async_copy(
                    ted_hbm.at[pl.ds(tok0, _T)], sc_v.at[b, 0], sem_in.at[b]),
                pltpu.make_async_copy(
                    tsumo_hbm.at[pl.ds(tok0, _T)], sc_v.at[b, 1], sem_in.at[b]),
                pltpu.make_async_copy(
                    ri_hbm.at[pl.ds(tok0, _T)], sc_v.at[b, 2], sem_in.at[b]),
                pltpu.make_async_copy(
                    called_hbm.at[pl.ds(tok0, _T)], sc_v.at[b, 3], sem_in.at[b]),
            ]

        def out_copy(c, b):
            tok0 = pl.multiple_of(wid * tok_per_w + c * _T, _T)
            return pltpu.make_async_copy(
                out_v.at[b], out_hbm.at[pl.ds(tok0, _T)], sem_out.at[b])

        cs = [bias_v[2, pl.ds(kk * _LANES, _LANES)]
              + bias_v[3, pl.ds(kk * _LANES, _LANES)] for kk in range(KD)]

        def fold(j, _):
            base = pl.multiple_of(j * D, D)
            for kk in range(KD):
                off = base + kk * _LANES
                table_v[pl.ds(off, _LANES)] = table_v[pl.ds(off, _LANES)] + cs[kk]
            return 0
        lax.fori_loop(0, V, fold, 0)

        tbs = [bias_v[0, pl.ds(kk * _LANES, _LANES)] for kk in range(KD)]
        rbs = [bias_v[1, pl.ds(kk * _LANES, _LANES)] for kk in range(KD)]

        for cp in in_copies(0, 0):
            cp.start()

        def pair(cc, _):
            for b in range(2):
                c = cc * 2 + b

                @pl.when(c + 1 < chunks)
                def _prefetch():
                    for cp in in_copies(c + 1, 1 - b):
                        cp.start()

                for cp in in_copies(c, b):
                    cp.wait()

                @pl.when(c >= 2)
                def _drain():
                    out_copy(c - 2, b).wait()

                @plsc.parallel_loop(0, _T // _LANES, 1, unroll=4)
                def _grp(g):
                    t0 = pl.multiple_of(g * _LANES, _LANES)
                    x16 = x_v[b, pl.ds(t0, _LANES)]
                    ted16 = sc_v[b, 0, pl.ds(t0, _LANES)]
                    ri16 = sc_v[b, 2, pl.ds(t0, _LANES)]
                    st16 = (sc_v[b, 1, pl.ds(t0, _LANES)]
                            + sc_v[b, 3, pl.ds(t0, _LANES)])
                    for lane in range(_LANES):
                        base = pl.multiple_of(x16[lane] * D, D)
                        tedt = ted16[lane]
                        rit = ri16[lane]
                        st = st16[lane]
                        rows = [table_v[pl.ds(base + kk * _LANES, _LANES)]
                                for kk in range(KD)]
                        vals = [(rows[kk] + st)
                                + (tedt * tbs[kk] + rit * rbs[kk])
                                for kk in range(KD)]
                        for kk in range(KD):
                            out_v[b, t0 + lane,
                                  pl.ds(kk * _LANES, _LANES)] = vals[kk]

                out_copy(c, b).start()
            return 0
        lax.fori_loop(0, chunks // 2, pair, 0)

        out_copy(chunks - 2, 0).wait()
        out_copy(chunks - 1, 1).wait()

    return k(x, ted, tsumo, ri, called, table, tb, rb, tsb, clb)


def _tc_embed(base, x, ted, tsumo, ri, called, table_pad,
              tb, rb, tsb, clb, N, NS, D):
    nb = (N - NS) // _TB
    blk0 = NS // _TB

    def body(base_ref, x_ref, ted_ref, tsumo_ref, ri_ref, called_ref,
             table_ref, tb_ref, rb_ref, tsb_ref, clb_ref, out_ref):
        ids = x_ref[0, 0, :]
        oh = (ids[:, None]
              == lax.broadcasted_iota(jnp.int32, (_TB, _VPAD), 1)
              ).astype(jnp.float32)
        emb = jnp.dot(oh, table_ref[...], preferred_element_type=jnp.float32)
        ted = ted_ref[0, 0, :][:, None]
        tsumo = tsumo_ref[0, 0, :][:, None]
        ri = ri_ref[0, 0, :][:, None]
        called = called_ref[0, 0, :][:, None]
        out_ref[...] = (emb + (tsumo + called)
                        + ted * tb_ref[...] + ri * rb_ref[...]
                        + (tsb_ref[...] + clb_ref[...]))

    tok_spec = pl.BlockSpec((1, 1, _TB), lambda i: (i, 0, 0))
    vec_spec = pl.BlockSpec((1, D), lambda i: (0, 0))
    return pl.pallas_call(
        body,
        grid=(nb,),
        in_specs=[
            pl.BlockSpec(memory_space=pl.ANY),
            tok_spec, tok_spec, tok_spec, tok_spec, tok_spec,
            pl.BlockSpec((_VPAD, D), lambda i: (0, 0)),
            vec_spec, vec_spec, vec_spec, vec_spec,
        ],
        out_specs=pl.BlockSpec((_TB, D), lambda i: (blk0 + i, 0)),
        out_shape=jax.ShapeDtypeStruct((N, D), jnp.float32),
        input_output_aliases={0: 0},
        compiler_params=pltpu.CompilerParams(
            dimension_semantics=("arbitrary",)),
    )(base, x, ted, tsumo, ri, called, table_pad, tb, rb, tsb, clb)


@functools.partial(jax.jit, static_argnums=(10, 11, 12, 13))
def _embed(x, ted, tsumo, ri, called, table, tb, rb, tsb, clb, N, NS, D, V):
    base = _sc_embed(x, ted, tsumo, ri, called, table.reshape(V * D),
                     tb.reshape(D), rb.reshape(D), tsb.reshape(D),
                     clb.reshape(D), N, NS, D, V)
    nb = (N - NS) // _TB
    table_pad = jnp.zeros((_VPAD, D), jnp.float32).at[:V].set(table)
    return _tc_embed(
        base,
        x[NS:].reshape(nb, 1, _TB),
        ted[NS:].reshape(nb, 1, _TB), tsumo[NS:].reshape(nb, 1, _TB),
        ri[NS:].reshape(nb, 1, _TB), called[NS:].reshape(nb, 1, _TB),
        table_pad, tb.reshape(1, D), rb.reshape(1, D),
        tsb.reshape(1, D), clb.reshape(1, D), N, NS, D)


def kernel(x, tedashi, tsumogiri, riichi, called, table,
           tedashi_bias, tsumogiri_bias, riichi_bias, called_bias):
    B, L = x.shape
    V, D = table.shape
    N = B * L
    NS = N // 2
    out = _embed(
        x.reshape(N).astype(jnp.int32),
        tedashi.reshape(N), tsumogiri.reshape(N),
        riichi.reshape(N), called.reshape(N),
        table,
        tedashi_bias.reshape(D), riichi_bias.reshape(D),
        tsumogiri_bias.reshape(D), called_bias.reshape(D),
        N, NS, D, V)
    return out.reshape(B, L, D)


# speedup vs baseline: 4.3007x; 1.2708x over previous
import functools

import jax
import jax.numpy as jnp
from jax import lax
from jax.experimental import pallas as pl
from jax.experimental.pallas import tpu as pltpu
from jax.experimental.pallas import tpu_sc as plsc

_LANES = 16
_NUM_WORKERS = 32
_T = 256


@functools.partial(jax.jit, static_argnums=(10, 11, 12))
def _sc_embed(x, ted, tsumo, ri, called, table, tb, rb, tsb, clb, N, D, V):
    KD = D // _LANES
    tok_per_w = N // _NUM_WORKERS
    chunks = tok_per_w // _T
    assert chunks % 2 == 0

    mesh = plsc.VectorSubcoreMesh(core_axis_name="c", subcore_axis_name="s")

    @functools.partial(
        pl.kernel,
        out_type=jax.ShapeDtypeStruct((N, D), jnp.float32),
        mesh=mesh,
        scratch_types=[
            pltpu.VMEM((V * D,), jnp.float32),
            pltpu.VMEM((4, D), jnp.float32),
            pltpu.VMEM((2, _T), jnp.int32),
            pltpu.VMEM((2, 4, _T), jnp.float32),
            pltpu.VMEM((2, _T, D), jnp.float32),
            pltpu.SemaphoreType.DMA((2,)),
            pltpu.SemaphoreType.DMA((2,)),
        ],
    )
    def k(x_hbm, ted_hbm, tsumo_hbm, ri_hbm, called_hbm, table_hbm,
          tb_hbm, rb_hbm, tsb_hbm, clb_hbm, out_hbm,
          table_v, bias_v, x_v, sc_v, out_v, sem_in, sem_out):
        wid = lax.axis_index("s") * 2 + lax.axis_index("c")

        pltpu.sync_copy(table_hbm, table_v)
        pltpu.sync_copy(tb_hbm, bias_v.at[0])
        pltpu.sync_copy(rb_hbm, bias_v.at[1])
        pltpu.sync_copy(tsb_hbm, bias_v.at[2])
        pltpu.sync_copy(clb_hbm, bias_v.at[3])

        def in_copies(c, b):
            tok0 = pl.multiple_of(wid * tok_per_w + c * _T, _T)
            return [
                pltpu.make_async_copy(
                    x_hbm.at[pl.ds(tok0, _T)], x_v.at[b], sem_in.at[b]),
                pltpu.make_async_copy(
                    ted_hbm.at[pl.ds(tok0, _T)], sc_v.at[b, 0], sem_in.at[b]),
                pltpu.make_async_copy(
                    tsumo_hbm.at[pl.ds(tok0, _T)], sc_v.at[b, 1], sem_in.at[b]),
                pltpu.make_async_copy(
                    ri_hbm.at[pl.ds(tok0, _T)], sc_v.at[b, 2], sem_in.at[b]),
                pltpu.make_async_copy(
                    called_hbm.at[pl.ds(tok0, _T)], sc_v.at[b, 3], sem_in.at[b]),
            ]

        def out_copy(c, b):
            tok0 = pl.multiple_of(wid * tok_per_w + c * _T, _T)
            return pltpu.make_async_copy(
                out_v.at[b], out_hbm.at[pl.ds(tok0, _T)], sem_out.at[b])

        cs = [bias_v[2, pl.ds(kk * _LANES, _LANES)]
              + bias_v[3, pl.ds(kk * _LANES, _LANES)] for kk in range(KD)]

        def fold(j, _):
            base = pl.multiple_of(j * D, D)
            for kk in range(KD):
                off = base + kk * _LANES
                table_v[pl.ds(off, _LANES)] = table_v[pl.ds(off, _LANES)] + cs[kk]
            return 0
        lax.fori_loop(0, V, fold, 0)

        tbs = [bias_v[0, pl.ds(kk * _LANES, _LANES)] for kk in range(KD)]
        rbs = [bias_v[1, pl.ds(kk * _LANES, _LANES)] for kk in range(KD)]

        for cp in in_copies(0, 0):
            cp.start()

        def pair(cc, _):
            for b in range(2):
                c = cc * 2 + b

                @pl.when(c + 1 < chunks)
                def _prefetch():
                    for cp in in_copies(c + 1, 1 - b):
                        cp.start()

                for cp in in_copies(c, b):
                    cp.wait()

                @pl.when(c >= 2)
                def _drain():
                    out_copy(c - 2, b).wait()

                @plsc.parallel_loop(0, _T // _LANES, 1, unroll=4)
                def _grp(g):
                    t0 = pl.multiple_of(g * _LANES, _LANES)
                    x16 = x_v[b, pl.ds(t0, _LANES)]
                    ted16 = sc_v[b, 0, pl.ds(t0, _LANES)]
                    ri16 = sc_v[b, 2, pl.ds(t0, _LANES)]
                    st16 = (sc_v[b, 1, pl.ds(t0, _LANES)]
                            + sc_v[b, 3, pl.ds(t0, _LANES)])
                    for lane in range(_LANES):
                        base = pl.multiple_of(x16[lane] * D, D)
                        tedt = ted16[lane]
                        rit = ri16[lane]
                        st = st16[lane]
                        rows = [table_v[pl.ds(base + kk * _LANES, _LANES)]
                                for kk in range(KD)]
                        vals = [(rows[kk] + st)
                                + (tedt * tbs[kk] + rit * rbs[kk])
                                for kk in range(KD)]
                        for kk in range(KD):
                            out_v[b, t0 + lane,
                                  pl.ds(kk * _LANES, _LANES)] = vals[kk]

                out_copy(c, b).start()
            return 0
        lax.fori_loop(0, chunks // 2, pair, 0)

        out_copy(chunks - 2, 0).wait()
        out_copy(chunks - 1, 1).wait()

    return k(x, ted, tsumo, ri, called, table, tb, rb, tsb, clb)


def kernel(x, tedashi, tsumogiri, riichi, called, table,
           tedashi_bias, tsumogiri_bias, riichi_bias, called_bias):
    B, L = x.shape
    V, D = table.shape
    N = B * L
    out = _sc_embed(
        x.reshape(N).astype(jnp.int32),
        tedashi.reshape(N), tsumogiri.reshape(N),
        riichi.reshape(N), called.reshape(N),
        table.reshape(V * D),
        tedashi_bias.reshape(D), riichi_bias.reshape(D),
        tsumogiri_bias.reshape(D), called_bias.reshape(D),
        N, D, V)
    return out.reshape(B, L, D)
